# baseline (speedup 1.0000x reference)
import numpy as np
import jax
import jax.numpy as jnp
from jax import lax
from jax.experimental import pallas as pl
from jax.experimental.pallas import tpu as pltpu
from jax.experimental.pallas import tpu_sc as plsc

_N_LABELS = 2048
_MAX_DEPTH = 4
_BATCH = 4096
_N_ROWS = 128
_NW = 32
_B_PER_W = _BATCH // _NW
_L = 16


def _build_onehot():
    a = np.zeros((_N_ROWS, 16), np.float32)
    r = 1
    for d in range(1, _MAX_DEPTH):
        for g in range(4 ** d):
            a[r, d - 1] = 1.0
            a[r, 4] = -float(g)
            r += 1
    return a


_A_CONST = _build_onehot()


def _tables_body(pt_ref, a_ref, t_ref, r_ref):
    pt = pt_ref[...]
    a = a_ref[...]
    s = jnp.dot(a, pt, preferred_element_type=jnp.float32)
    t_ref[...] = s == 0.0
    d = lax.broadcasted_iota(jnp.int32, (8, _N_LABELS), 0)
    rr = jnp.where(d == 1, pt[0:1, :] + 1.0,
         jnp.where(d == 2, pt[1:2, :] + 5.0,
         jnp.where(d == 3, pt[2:3, :] + 21.0, 0.0)))
    rr = jnp.clip(rr, 0.0, float(_N_ROWS - 1))
    r_ref[...] = rr.astype(jnp.int32)


_build_tables = pl.pallas_call(
    _tables_body,
    out_shape=(
        jax.ShapeDtypeStruct((_N_ROWS, _N_LABELS), jnp.bool_),
        jax.ShapeDtypeStruct((8, _N_LABELS), jnp.int32),
    ),
)


def _sc_body(t_hbm, r_hbm, yl_hbm, d_hbm, out_hbm,
             rof_v, yl_v, d_v, rows_v, sem0, sem1):
    wid = lax.axis_index("s") * 2 + lax.axis_index("c")
    base = wid * _B_PER_W
    pltpu.sync_copy(r_hbm, rof_v)
    pltpu.sync_copy(yl_hbm.at[pl.ds(base, _B_PER_W)], yl_v)
    pltpu.sync_copy(d_hbm.at[pl.ds(base, _B_PER_W)], d_v)

    def ridx(k):
        dv = d_v[pl.ds(k * _L, _L)]
        iv = yl_v[pl.ds(k * _L, _L)]
        return plsc.load_gather(rof_v, [(dv << 11) + iv])

    n = _B_PER_W // _L
    sems = (sem0, sem1)
    descs = [None] * n
    descs[0] = pltpu.async_copy(t_hbm.at[ridx(0)], rows_v.at[0], sem0)
    for k in range(n):
        if k + 1 < n:
            descs[k + 1] = pltpu.async_copy(
                t_hbm.at[ridx(k + 1)], rows_v.at[(k + 1) & 1], sems[(k + 1) & 1])
        descs[k].wait()
        pltpu.sync_copy(rows_v.at[k & 1],
                        out_hbm.at[pl.ds(base + k * _L, _L)])


_SC_GATHER = None


def _sc_gather():
    global _SC_GATHER
    if _SC_GATHER is None:
        _SC_GATHER = pl.kernel(
            _sc_body,
            out_type=jax.ShapeDtypeStruct((_BATCH, _N_LABELS), jnp.bool_),
            mesh=plsc.VectorSubcoreMesh(core_axis_name="c",
                                        subcore_axis_name="s"),
            scratch_types=[
                pltpu.VMEM((8 * _N_LABELS,), jnp.int32),
                pltpu.VMEM((_B_PER_W,), jnp.int32),
                pltpu.VMEM((_B_PER_W,), jnp.int32),
                pltpu.VMEM((2, _L, _N_LABELS), jnp.bool_),
                pltpu.SemaphoreType.DMA,
                pltpu.SemaphoreType.DMA,
            ],
            compiler_params=pltpu.CompilerParams(needs_layout_passes=False),
        )
    return _SC_GATHER


def kernel(y, depths, label_pool):
    yl = y[:, -1].astype(jnp.int32)
    dd = depths[:, 0].astype(jnp.int32)
    pt = jnp.concatenate([
        label_pool.T.astype(jnp.float32),
        jnp.ones((1, _N_LABELS), jnp.float32),
        jnp.zeros((16 - _MAX_DEPTH - 1, _N_LABELS), jnp.float32),
    ], axis=0)
    t, r = _build_tables(pt, jnp.asarray(_A_CONST))
    return _sc_gather()(t, r.reshape(-1), yl, dd)
